# Initial kernel scaffold; baseline (speedup 1.0000x reference)
import functools
import math

import jax
import jax.numpy as jnp
from jax import lax
from jax.experimental import pallas as pl
from jax.experimental.pallas import tpu as pltpu
from jax.experimental.pallas import tpu_sc as plsc

EMB = 64
SCALE = math.sqrt(EMB)

_NC = 2
_NS = 16
_NW = _NC * _NS

_CHUNK = 800


def _tile_body(table_hbm, x_hbm, out_hbm, idx_v, rows_v, sem, *, b_per_w,
               n_chunks):
  wid = lax.axis_index("s") * _NC + lax.axis_index("c")
  base = wid * b_per_w

  def chunk(i, carry):
    off = base + i * _CHUNK
    pltpu.sync_copy(x_hbm.at[pl.ds(off, _CHUNK)], idx_v)
    pltpu.async_copy(table_hbm.at[idx_v], rows_v, sem).wait()

    def scale_row(r, c):
      for j in range(EMB // 16):
        rows_v[r, pl.ds(j * 16, 16)] = rows_v[r, pl.ds(j * 16, 16)] * SCALE
      return c

    lax.fori_loop(0, _CHUNK, scale_row, 0)
    pltpu.sync_copy(rows_v, out_hbm.at[pl.ds(off, _CHUNK)])
    return carry

  lax.fori_loop(0, n_chunks, chunk, 0)


@jax.jit
def _lookup(table, xf):
  b = xf.shape[0]
  assert b % _NW == 0
  b_per_w = b // _NW
  assert b_per_w % _CHUNK == 0
  n_chunks = b_per_w // _CHUNK

  mesh = plsc.VectorSubcoreMesh(core_axis_name="c", subcore_axis_name="s")
  return pl.kernel(
      functools.partial(_tile_body, b_per_w=b_per_w, n_chunks=n_chunks),
      out_type=jax.ShapeDtypeStruct((b, EMB), jnp.float32),
      mesh=mesh,
      scratch_types=[
          pltpu.VMEM((_CHUNK,), jnp.int32),
          pltpu.VMEM((_CHUNK, EMB), jnp.float32),
          pltpu.SemaphoreType.DMA,
      ],
  )(table, xf)


def kernel(x, table):
  s, t = x.shape
  out = _lookup(table, x.reshape(s * t))
  return out.reshape(s, t, EMB)


# speedup vs baseline: 1.0333x; 1.0333x over previous
import functools
import math

import jax
import jax.numpy as jnp
from jax import lax
from jax.experimental import pallas as pl
from jax.experimental.pallas import tpu as pltpu
from jax.experimental.pallas import tpu_sc as plsc

EMB = 64
SCALE = math.sqrt(EMB)

_NC = 2
_NS = 16
_NW = _NC * _NS

_CHUNK = 800


def _tile_body(table_hbm, x_hbm, out_hbm, idx_v, rows_v, sem, *, b_per_w,
               n_chunks):
  wid = lax.axis_index("s") * _NC + lax.axis_index("c")
  base = wid * b_per_w

  def chunk(i, carry):
    off = base + i * _CHUNK
    pltpu.sync_copy(x_hbm.at[pl.ds(off, _CHUNK)], idx_v)
    pltpu.async_copy(table_hbm.at[idx_v], rows_v, sem).wait()

    def scale_row(r, c):
      for j in range(EMB // 16):
        rows_v[r, pl.ds(j * 16, 16)] = rows_v[r, pl.ds(j * 16, 16)] * SCALE
      return c

    lax.fori_loop(0, _CHUNK, scale_row, 0)
    pltpu.sync_copy(rows_v, out_hbm.at[pl.ds(off, _CHUNK)])
    return carry

  lax.fori_loop(0, n_chunks, chunk, 0)


@jax.jit
def _lookup(table, xf):
  b = xf.shape[0]
  assert b % _NW == 0
  b_per_w = b // _NW
  assert b_per_w % _CHUNK == 0
  n_chunks = b_per_w // _CHUNK

  mesh = plsc.VectorSubcoreMesh(core_axis_name="c", subcore_axis_name="s")
  return pl.kernel(
      functools.partial(_tile_body, b_per_w=b_per_w, n_chunks=n_chunks),
      out_type=jax.ShapeDtypeStruct((b, EMB), jnp.float32),
      mesh=mesh,
      scratch_types=[
          pltpu.VMEM((_CHUNK,), jnp.int32),
          pltpu.VMEM((_CHUNK, EMB), jnp.float32),
          pltpu.SemaphoreType.DMA,
      ],
      compiler_params=pltpu.CompilerParams(use_tc_tiling_on_sc=False),
  )(table, xf)


def kernel(x, table):
  s, t = x.shape
  out = _lookup(table, x.reshape(s * t))
  return out.reshape(s, t, EMB)
